# baseline (speedup 1.0000x reference)
import functools
import math

import jax
import jax.numpy as jnp
from jax import lax
from jax.experimental import pallas as pl
from jax.experimental.pallas import tpu as pltpu
from jax.experimental.pallas import tpu_sc as plsc

D_MODEL = 64
D_PAD = 128
VOCAB = 1000000
B_TOTAL = 4096 * 200
NC, NS = 2, 16
NW = NC * NS
ROWS_PER_W = B_TOTAL // NW
SUB = 1
CHUNK = SUB * 128
N_CHUNKS = ROWS_PER_W // CHUNK
NBUF = 2
SCALE = math.sqrt(D_MODEL)

_mesh = plsc.VectorSubcoreMesh(
    core_axis_name="c", subcore_axis_name="s", num_cores=NC, num_subcores=NS
)

NFULL = VOCAB // 128
TAIL = VOCAB - NFULL * 128
BASE_BLK = NFULL // NW
EXTRA = NFULL - BASE_BLK * NW


@functools.partial(
    pl.kernel,
    out_type=jax.ShapeDtypeStruct((VOCAB // 2, 128), jnp.float32),
    mesh=_mesh,
    scratch_types=[
        pltpu.VMEM((2, D_MODEL, 128), jnp.float32),
        pltpu.VMEM((2, 64, 128), jnp.float32),
        pltpu.VMEM((TAIL, D_MODEL), jnp.float32),
        pltpu.SemaphoreType.DMA,
        pltpu.SemaphoreType.DMA,
        pltpu.SemaphoreType.DMA,
        pltpu.SemaphoreType.DMA,
    ],
    compiler_params=pltpu.CompilerParams(
        use_tc_tiling_on_sc=True, needs_layout_passes=False),
)
def _transpose_lut(lt_hbm, tail_hbm, t_hbm, src_v, dst_v, tail_v, sl0, sl1, st0, st1):
    sl = (sl0, sl1)
    st = (st0, st1)
    wid = lax.axis_index("s") * NC + lax.axis_index("c")
    start = wid * BASE_BLK + lax.min(wid, EXTRA)
    nblk = BASE_BLK + jnp.where(wid < EXTRA, 1, 0)
    lanes = jax.lax.iota(jnp.int32, 16)

    def ld_start(c, b):
        pltpu.async_copy(
            lt_hbm.at[:, pl.ds(c * 128, 128)], src_v.at[b], sl[b])

    def ld_wait(b):
        pltpu.make_async_copy(
            lt_hbm.at[:, pl.ds(0, 128)], src_v.at[b], sl[b]).wait()

    def st_start(c, b):
        pltpu.async_copy(
            dst_v.at[b], t_hbm.at[pl.ds(c * 64, 64)], st[b])

    def st_wait(b):
        pltpu.make_async_copy(
            dst_v.at[b], t_hbm.at[pl.ds(0, 64)], st[b]).wait()

    def trans_buf(b):
        sv = src_v.at[b]
        dv = dst_v.at[b]

        def trow(jp):
            for j in range(8):
                cvec = jnp.full((16,), 2 * jp + (j // 4), jnp.int32)
                v = plsc.load_gather(sv, [lanes + 16 * (j % 4), cvec])
                dv[jp, pl.ds(16 * j, 16)] = v

        plsc.parallel_loop(0, 64, unroll=4)(trow)

    ld_start(start, 0)

    def blk(it, carry):
        g = it * 2
        for b in range(2):
            k = g + b

            def body(b=b, k=k):
                ld_wait(b)
                pl.when(k + 1 < nblk)(lambda: ld_start(start + k + 1, b ^ 1))
                pl.when(k >= 2)(lambda: st_wait(b))
                trans_buf(b)
                st_start(start + k, b)

            pl.when(k < nblk)(body)
        return carry

    lax.fori_loop(0, (BASE_BLK + 2) // 2, blk, 0)
    st_wait(0)
    st_wait(1)

    @pl.when(wid == 0)
    def _tail():
        pltpu.sync_copy(tail_hbm, tail_v)

        def trow(jp, c2):
            for j in range(8):
                rvec = jnp.full((16,), 2 * jp + (j // 4), jnp.int32)
                v = plsc.load_gather(tail_v, [rvec, lanes + 16 * (j % 4)])
                dst_v[0, jp, pl.ds(16 * j, 16)] = v
            return c2

        lax.fori_loop(0, TAIL // 2, trow, 0)
        pltpu.sync_copy(dst_v.at[0, pl.ds(0, TAIL // 2)],
                        t_hbm.at[pl.ds(NFULL * 64, TAIL // 2)])


@functools.partial(
    pl.kernel,
    out_type=jax.ShapeDtypeStruct((B_TOTAL // 2, 128), jnp.float32),
    mesh=_mesh,
    scratch_types=[
        pltpu.VMEM((NBUF, SUB, 128), jnp.int32),
        pltpu.VMEM((NBUF, CHUNK, D_MODEL), jnp.float32),
        pltpu.VMEM((NBUF, CHUNK // 2, 128), jnp.float32),
        pltpu.SemaphoreType.DMA,
        pltpu.SemaphoreType.DMA,
        pltpu.SemaphoreType.DMA,
        pltpu.SemaphoreType.DMA,
        pltpu.SemaphoreType.DMA,
        pltpu.SemaphoreType.DMA,
    ],
    compiler_params=pltpu.CompilerParams(
        use_tc_tiling_on_sc=False, needs_layout_passes=False),
)
def _embed_gather(lut_hbm, idx_hbm, out_hbm, idx_v, rows_v, outs_v,
                  si0, si1, sg0, sg1, ss0, ss1):
    si = (si0, si1)
    sg = (sg0, sg1)
    ss = (ss0, ss1)
    wid = lax.axis_index("s") * NC + lax.axis_index("c")
    irow0 = wid * (ROWS_PER_W // 128)
    row0 = wid * (ROWS_PER_W // 2)

    def idx_start(g, b):
        pltpu.async_copy(
            idx_hbm.at[pl.ds(irow0 + g * SUB, SUB)], idx_v.at[b], si[b])

    def idx_wait(b):
        pltpu.make_async_copy(
            idx_hbm.at[pl.ds(irow0, SUB)], idx_v.at[b], si[b]).wait()

    def gat_start(b):
        for j in range(SUB):
            pltpu.async_copy(
                lut_hbm.at[idx_v.at[b, j]],
                rows_v.at[b, pl.ds(j * 128, 128)], sg[b])

    def gat_wait(b):
        for j in range(SUB):
            pltpu.make_async_copy(
                lut_hbm.at[idx_v.at[b, j]],
                rows_v.at[b, pl.ds(j * 128, 128)], sg[b]).wait()

    def store_start(g, b):
        pltpu.async_copy(
            outs_v.at[b],
            out_hbm.at[pl.ds(row0 + g * (CHUNK // 2), CHUNK // 2)], ss[b])

    def store_wait(b):
        pltpu.make_async_copy(
            outs_v.at[b], out_hbm.at[pl.ds(row0, CHUNK // 2)], ss[b]).wait()

    def scale_buf(b):
        rv = rows_v.at[b]
        ov = outs_v.at[b]

        def srow(m):
            for j in range(8):
                ov[m, pl.ds(16 * j, 16)] = (
                    rv[2 * m + (j // 4), pl.ds(16 * (j % 4), 16)] * SCALE)

        plsc.parallel_loop(0, CHUNK // 2, unroll=8)(srow)

    idx_start(0, 0)
    idx_start(1, 1)
    idx_wait(0)
    gat_start(0)

    def step(it, carry):
        for b in range(NBUF):
            g = it * NBUF + b
            o = b ^ 1
            gat_wait(b)
            pl.when(g < N_CHUNKS - NBUF)(lambda: idx_start(g + NBUF, b))
            pl.when(g >= 1)(lambda: store_wait(o))
            def fire_next():
                idx_wait(o)
                gat_start(o)
            pl.when(g < N_CHUNKS - 1)(fire_next)
            scale_buf(b)
            store_start(g, b)
        return carry

    lax.fori_loop(0, N_CHUNKS // NBUF, step, 0)

    store_wait((N_CHUNKS - 1) % NBUF)


def kernel(x, lut):
    t_pairs = _transpose_lut(lut.T, lut[NFULL * 128:, :])
    t_rows = t_pairs.reshape(VOCAB, D_MODEL)
    idx2 = x.reshape(B_TOTAL // 128, 128).astype(jnp.int32)
    out = _embed_gather(t_rows, idx2)
    return out.reshape(x.shape[0], x.shape[1], D_MODEL)


# speedup vs baseline: 1.4331x; 1.4331x over previous
import functools
import math

import jax
import jax.numpy as jnp
from jax import lax
from jax.experimental import pallas as pl
from jax.experimental.pallas import tpu as pltpu
from jax.experimental.pallas import tpu_sc as plsc

D_MODEL = 64
D_PAD = 128
VOCAB = 1000000
B_TOTAL = 4096 * 200
NC, NS = 2, 16
NW = NC * NS
ROWS_PER_W = B_TOTAL // NW
SUB = 1
CHUNK = SUB * 128
N_CHUNKS = ROWS_PER_W // CHUNK
NBUF = 2
SCALE = math.sqrt(D_MODEL)

_mesh = plsc.VectorSubcoreMesh(
    core_axis_name="c", subcore_axis_name="s", num_cores=NC, num_subcores=NS
)


@functools.partial(
    pl.kernel,
    out_type=jax.ShapeDtypeStruct((B_TOTAL, D_MODEL), jnp.float32),
    mesh=_mesh,
    scratch_types=[
        pltpu.VMEM((NBUF, SUB, 128), jnp.int32),
        pltpu.VMEM((NBUF, CHUNK, D_PAD), jnp.float32),
        pltpu.VMEM((NBUF, CHUNK, D_MODEL), jnp.float32),
        pltpu.SemaphoreType.DMA,
        pltpu.SemaphoreType.DMA,
        pltpu.SemaphoreType.DMA,
        pltpu.SemaphoreType.DMA,
        pltpu.SemaphoreType.DMA,
        pltpu.SemaphoreType.DMA,
    ],
    compiler_params=pltpu.CompilerParams(use_tc_tiling_on_sc=True),
)
def _embed_gather(lut_hbm, idx_hbm, out_hbm, idx_v, rows_v, outs_v,
                  si0, si1, sg0, sg1, ss0, ss1):
    si = (si0, si1)
    sg = (sg0, sg1)
    ss = (ss0, ss1)
    wid = lax.axis_index("s") * NC + lax.axis_index("c")
    irow0 = wid * (ROWS_PER_W // 128)
    row0 = wid * ROWS_PER_W

    def idx_start(g, b):
        pltpu.async_copy(
            idx_hbm.at[pl.ds(irow0 + g * SUB, SUB)], idx_v.at[b], si[b])

    def idx_wait(b):
        pltpu.make_async_copy(
            idx_hbm.at[pl.ds(irow0, SUB)], idx_v.at[b], si[b]).wait()

    def gat_start(b):
        for j in range(SUB):
            pltpu.async_copy(
                lut_hbm.at[idx_v.at[b, j]],
                rows_v.at[b, pl.ds(j * 128, 128)], sg[b])

    def gat_wait(b):
        for j in range(SUB):
            pltpu.make_async_copy(
                lut_hbm.at[idx_v.at[b, j]],
                rows_v.at[b, pl.ds(j * 128, 128)], sg[b]).wait()

    def store_start(g, b):
        pltpu.async_copy(
            outs_v.at[b], out_hbm.at[pl.ds(row0 + g * CHUNK, CHUNK)], ss[b])

    def store_wait(b):
        pltpu.make_async_copy(
            outs_v.at[b], out_hbm.at[pl.ds(row0, CHUNK)], ss[b]).wait()

    def scale_buf(b):
        rv = rows_v.at[b]
        ov = outs_v.at[b]

        def scale_blk(k, c):
            base = k * 4
            for u in range(4):
                for j in range(D_MODEL // 16):
                    sl = pl.ds(j * 16, 16)
                    ov[base + u, sl] = rv[base + u, sl] * SCALE
            return c

        lax.fori_loop(0, CHUNK // 4, scale_blk, 0)

    idx_start(0, 0)
    idx_start(1, 1)
    idx_wait(0)
    gat_start(0)

    def step(it, carry):
        for b in range(NBUF):
            g = it * NBUF + b
            o = b ^ 1
            gat_wait(b)
            pl.when(g < N_CHUNKS - NBUF)(lambda: idx_start(g + NBUF, b))
            pl.when(g >= 1)(lambda: store_wait(o))
            def fire_next():
                idx_wait(o)
                gat_start(o)
            pl.when(g < N_CHUNKS - 1)(fire_next)
            scale_buf(b)
            store_start(g, b)
        return carry

    lax.fori_loop(0, N_CHUNKS // NBUF, step, 0)

    store_wait((N_CHUNKS - 1) % NBUF)


def kernel(x, lut):
    lut_p = jnp.pad(lut, ((0, 0), (0, D_PAD - D_MODEL)))
    idx2 = x.reshape(B_TOTAL // 128, 128).astype(jnp.int32)
    out = _embed_gather(lut_p, idx2)
    return out.reshape(x.shape[0], x.shape[1], D_MODEL)
